# Initial kernel scaffold; baseline (speedup 1.0000x reference)
import jax
import jax.numpy as jnp
from jax.experimental import pallas as pl


def kernel(z, edge_index, dist, node_table, edge_table, W_n1, b_n1, W_e1, b_e1, W_l1, b_l1, W_l2, b_l2, W_l3, b_l3, W_n2, b_n2, W_n3, b_n3, W_r1, b_r1, W_r2, b_r2):
    raise NotImplementedError("write your pallas kernel here")


# speedup vs baseline: 1.0734x; 1.0734x over previous
import functools

import jax
import jax.numpy as jnp
from jax import lax
from jax.experimental import pallas as pl

N_TYPES = 100
RBF_DIM = 30
DIM = 64
N_CONV = 3
GAP = 10.0 / 29.0


def _sp(x, beta, thr):
    xb = x * beta
    return jnp.where(xb > thr, x, (1.0 / beta) * jnp.log1p(jnp.exp(jnp.minimum(xb, thr))))


def _t_body(dist_ref, wl1_ref, bl1_ref, w32_ref, b32_ref, tA_ref, tB_ref):
    d = dist_ref[...]
    j = lax.broadcasted_iota(jnp.int32, (1, RBF_DIM), 1).astype(jnp.float32)
    x = d - j * GAP
    rbf = jnp.exp(-(x * x) * (1.0 / GAP))
    u = jnp.dot(rbf, wl1_ref[0], preferred_element_type=jnp.float32) + bl1_ref[0, 0]
    u = _sp(u, 0.5, 14.0)
    t = jnp.dot(u, w32_ref[0], preferred_element_type=jnp.float32) + b32_ref[0, 0]
    tA_ref[0] = t[:, :32]
    tB_ref[0] = t[:, 32:]


def _edge_t(dist, W_l1T, b_l1, W32T, b32, BE):
    E = dist.shape[0]
    nb = E // BE
    return pl.pallas_call(
        _t_body,
        grid=(N_CONV, nb),
        in_specs=[
            pl.BlockSpec((BE, 1), lambda l, i: (i, 0)),
            pl.BlockSpec((1, RBF_DIM, DIM), lambda l, i: (l, 0, 0)),
            pl.BlockSpec((1, 1, DIM), lambda l, i: (l, 0, 0)),
            pl.BlockSpec((1, DIM, DIM), lambda l, i: (l, 0, 0)),
            pl.BlockSpec((1, 1, DIM), lambda l, i: (l, 0, 0)),
        ],
        out_specs=[
            pl.BlockSpec((1, BE, 32), lambda l, i: (l, i, 0)),
            pl.BlockSpec((1, BE, 32), lambda l, i: (l, i, 0)),
        ],
        out_shape=[
            jax.ShapeDtypeStruct((N_CONV, E, 32), jnp.float32),
            jax.ShapeDtypeStruct((N_CONV, E, 32), jnp.float32),
        ],
    )(dist.reshape(E, 1), W_l1T, b_l1.reshape(N_CONV, 1, DIM),
      W32T, b32.reshape(N_CONV, 1, DIM))


def _init_body(z_ref, tbl_ref, wr1_ref, wn1_ref, bn1_ref,
               h0_ref, racc_ref, hnA_ref, hnB_ref):
    zb = z_ref[...]
    ids = lax.broadcasted_iota(jnp.int32, (zb.shape[0], N_TYPES), 1)
    oh = (ids == zb).astype(jnp.float32)
    h0 = jnp.dot(oh, tbl_ref[...], preferred_element_type=jnp.float32)
    h0_ref[...] = h0
    racc_ref[...] = jnp.dot(h0, wr1_ref[...], preferred_element_type=jnp.float32)
    hn = jnp.dot(h0, wn1_ref[...], preferred_element_type=jnp.float32) + bn1_ref[0]
    hnA_ref[...] = hn[:, :32]
    hnB_ref[...] = hn[:, 32:]


def _node_init(z, node_table, W_r1T_0, W_n1T_0, b_n1_0, BN):
    N = z.shape[0]
    nb = N // BN
    return pl.pallas_call(
        _init_body,
        grid=(nb,),
        in_specs=[
            pl.BlockSpec((BN, 1), lambda i: (i, 0)),
            pl.BlockSpec((N_TYPES, DIM), lambda i: (0, 0)),
            pl.BlockSpec((DIM, DIM), lambda i: (0, 0)),
            pl.BlockSpec((DIM, DIM), lambda i: (0, 0)),
            pl.BlockSpec((1, DIM), lambda i: (0, 0)),
        ],
        out_specs=[
            pl.BlockSpec((BN, DIM), lambda i: (i, 0)),
            pl.BlockSpec((BN, DIM), lambda i: (i, 0)),
            pl.BlockSpec((BN, 32), lambda i: (i, 0)),
            pl.BlockSpec((BN, 32), lambda i: (i, 0)),
        ],
        out_shape=[
            jax.ShapeDtypeStruct((N, DIM), jnp.float32),
            jax.ShapeDtypeStruct((N, DIM), jnp.float32),
            jax.ShapeDtypeStruct((N, 32), jnp.float32),
            jax.ShapeDtypeStruct((N, 32), jnp.float32),
        ],
    )(z.reshape(N, 1), node_table, W_r1T_0, W_n1T_0, b_n1_0)


def _lvl_body(nA_ref, nB_ref, hp_ref, rin_ref, wn2_ref, bn2_ref, wn3_ref,
              bn3_ref, wr1_ref, wn1_ref, bn1_ref,
              h_ref, racc_ref, hnA_ref, hnB_ref):
    node = jnp.concatenate([nA_ref[...], nB_ref[...]], axis=1)
    x1 = jnp.dot(node, wn2_ref[...], preferred_element_type=jnp.float32) + bn2_ref[0]
    a = _sp(x1, 0.5, 14.0)
    h = hp_ref[...] + jnp.dot(a, wn3_ref[...], preferred_element_type=jnp.float32) + bn3_ref[0]
    h_ref[...] = h
    racc_ref[...] = rin_ref[...] + jnp.dot(h, wr1_ref[...], preferred_element_type=jnp.float32)
    hn = jnp.dot(h, wn1_ref[...], preferred_element_type=jnp.float32) + bn1_ref[0]
    hnA_ref[...] = hn[:, :32]
    hnB_ref[...] = hn[:, 32:]


def _node_level(nodeA, nodeB, h_prev, racc, wn2T, bn2, wn3T, bn3,
                wr1T_next, wn1T_next, bn1_next, BN):
    N = h_prev.shape[0]
    nb = N // BN
    wspec = pl.BlockSpec((DIM, DIM), lambda i: (0, 0))
    bspec = pl.BlockSpec((1, DIM), lambda i: (0, 0))
    half = pl.BlockSpec((BN, 32), lambda i: (i, 0))
    full = pl.BlockSpec((BN, DIM), lambda i: (i, 0))
    return pl.pallas_call(
        _lvl_body,
        grid=(nb,),
        in_specs=[half, half, full, full, wspec, bspec, wspec, bspec,
                  wspec, wspec, bspec],
        out_specs=[full, full, half, half],
        out_shape=[
            jax.ShapeDtypeStruct((N, DIM), jnp.float32),
            jax.ShapeDtypeStruct((N, DIM), jnp.float32),
            jax.ShapeDtypeStruct((N, 32), jnp.float32),
            jax.ShapeDtypeStruct((N, 32), jnp.float32),
        ],
    )(nodeA, nodeB, h_prev, racc, wn2T, bn2, wn3T, bn3,
      wr1T_next, wn1T_next, bn1_next)


def _read_body(racc_ref, br1_ref, wr2_ref, o_ref):
    x = racc_ref[...] + br1_ref[0]
    h = _sp(x, 1.0, 20.0)
    p = jnp.sum(h * wr2_ref[...]).reshape(1, 1)

    @pl.when(pl.program_id(0) == 0)
    def _():
        o_ref[...] = jnp.zeros((1, 1), jnp.float32)

    o_ref[...] += p


def _readout(racc, b_r1, W_r2, BN):
    N = racc.shape[0]
    nb = N // BN
    return pl.pallas_call(
        _read_body,
        grid=(nb,),
        in_specs=[
            pl.BlockSpec((BN, DIM), lambda i: (i, 0)),
            pl.BlockSpec((1, DIM), lambda i: (0, 0)),
            pl.BlockSpec((1, DIM), lambda i: (0, 0)),
        ],
        out_specs=pl.BlockSpec((1, 1), lambda i: (0, 0)),
        out_shape=jax.ShapeDtypeStruct((1, 1), jnp.float32),
    )(racc, b_r1, W_r2)


def kernel(z, edge_index, dist, node_table, edge_table,
           W_n1, b_n1, W_e1, b_e1, W_l1, b_l1, W_l2, b_l2, W_l3, b_l3,
           W_n2, b_n2, W_n3, b_n3, W_r1, b_r1, W_r2, b_r2):
    N = z.shape[0]
    E = dist.shape[0]
    BE = 2000 if E % 2000 == 0 else E
    BN = 2000 if N % 2000 == 0 else N

    src = edge_index[0]
    dst = edge_index[1]

    W32 = jnp.einsum('lij,ljk->lik', W_l3, W_l2)
    b32 = jnp.einsum('lij,lj->li', W_l3, b_l2) + b_l3
    W_l1T = W_l1.transpose(0, 2, 1)
    W32T = W32.transpose(0, 2, 1)
    W_n1T = W_n1.transpose(0, 2, 1)
    W_n2T = W_n2.transpose(0, 2, 1)
    W_n3T = W_n3.transpose(0, 2, 1)
    W_r1T = W_r1.T

    tA, tB = _edge_t(dist, W_l1T, b_l1, W32T, b32, BE)

    h, racc, hnA, hnB = _node_init(
        z, node_table, W_r1T[:DIM], W_n1T[0], b_n1[0:1], BN)

    for l in range(N_CONV):
        t_l = jnp.concatenate([tA[l], tB[l]], axis=1)
        hn = jnp.concatenate([hnA, hnB], axis=1)
        m = (hn[src] + 1.0) * t_l
        node = jnp.zeros((N, DIM), jnp.float32).at[dst].add(m)
        nxt = (l + 1) % N_CONV
        h, racc, hnA, hnB = _node_level(
            node[:, :32], node[:, 32:], h, racc,
            W_n2T[l], b_n2[l:l + 1], W_n3T[l], b_n3[l:l + 1],
            W_r1T[DIM * (l + 1):DIM * (l + 2)], W_n1T[nxt], b_n1[nxt:nxt + 1],
            BN)

    out = _readout(racc, b_r1.reshape(1, DIM), W_r2, BN)
    return (out[0] + N * b_r2[0]).reshape(1)


# speedup vs baseline: 1.6855x; 1.5703x over previous
import functools

import jax
import jax.numpy as jnp
from jax import lax
from jax.experimental import pallas as pl
from jax.experimental.pallas import tpu as pltpu
from jax.experimental.pallas import tpu_sc as plsc

N_TYPES = 100
RBF_DIM = 30
DIM = 64
HALF = 32
N_CONV = 3
GAP = 10.0 / 29.0

NC = 2
NS = 16
LANES = 16
CH = 80


def _sp(x, beta, thr):
    xb = x * beta
    return jnp.where(xb > thr, x, (1.0 / beta) * jnp.log1p(jnp.exp(jnp.minimum(xb, thr))))


def _t_body(dist_ref, wl1_ref, bl1_ref, w32_ref, b32_ref, t_ref):
    d = dist_ref[...]
    j = lax.broadcasted_iota(jnp.int32, (1, RBF_DIM), 1).astype(jnp.float32)
    x = d - j * GAP
    rbf = jnp.exp(-(x * x) * (1.0 / GAP))
    u = jnp.dot(rbf, wl1_ref[0], preferred_element_type=jnp.float32) + bl1_ref[0, 0]
    u = _sp(u, 0.5, 14.0)
    t = jnp.dot(u, w32_ref[0], preferred_element_type=jnp.float32) + b32_ref[0, 0]
    t_ref[0, 0] = t[:, :HALF]
    t_ref[0, 1] = t[:, HALF:]


def _edge_t(dist, W_l1T, b_l1, W32T, b32, BE):
    E = dist.shape[0]
    nb = E // BE
    return pl.pallas_call(
        _t_body,
        grid=(N_CONV, nb),
        in_specs=[
            pl.BlockSpec((BE, 1), lambda l, i: (i, 0)),
            pl.BlockSpec((1, RBF_DIM, DIM), lambda l, i: (l, 0, 0)),
            pl.BlockSpec((1, 1, DIM), lambda l, i: (l, 0, 0)),
            pl.BlockSpec((1, DIM, DIM), lambda l, i: (l, 0, 0)),
            pl.BlockSpec((1, 1, DIM), lambda l, i: (l, 0, 0)),
        ],
        out_specs=pl.BlockSpec((1, NC, BE, HALF), lambda l, i: (l, 0, i, 0)),
        out_shape=jax.ShapeDtypeStruct((N_CONV, NC, E, HALF), jnp.float32),
    )(dist.reshape(E, 1), W_l1T, b_l1.reshape(N_CONV, 1, DIM),
      W32T, b32.reshape(N_CONV, 1, DIM))


def _init_body(z_ref, tbl_ref, wr1_ref, wn1_ref, bn1_ref,
               h0_ref, racc_ref, hn_ref):
    zb = z_ref[...]
    ids = lax.broadcasted_iota(jnp.int32, (zb.shape[0], N_TYPES), 1)
    oh = (ids == zb).astype(jnp.float32)
    h0 = jnp.dot(oh, tbl_ref[...], preferred_element_type=jnp.float32)
    h0_ref[...] = h0
    racc_ref[...] = jnp.dot(h0, wr1_ref[...], preferred_element_type=jnp.float32)
    hn = jnp.dot(h0, wn1_ref[...], preferred_element_type=jnp.float32) + bn1_ref[0]
    hn_ref[0] = hn[:, :HALF]
    hn_ref[1] = hn[:, HALF:]


def _node_init(z, node_table, W_r1T_0, W_n1T_0, b_n1_0, BN):
    N = z.shape[0]
    nb = N // BN
    return pl.pallas_call(
        _init_body,
        grid=(nb,),
        in_specs=[
            pl.BlockSpec((BN, 1), lambda i: (i, 0)),
            pl.BlockSpec((N_TYPES, DIM), lambda i: (0, 0)),
            pl.BlockSpec((DIM, DIM), lambda i: (0, 0)),
            pl.BlockSpec((DIM, DIM), lambda i: (0, 0)),
            pl.BlockSpec((1, DIM), lambda i: (0, 0)),
        ],
        out_specs=[
            pl.BlockSpec((BN, DIM), lambda i: (i, 0)),
            pl.BlockSpec((BN, DIM), lambda i: (i, 0)),
            pl.BlockSpec((NC, BN, HALF), lambda i: (0, i, 0)),
        ],
        out_shape=[
            jax.ShapeDtypeStruct((N, DIM), jnp.float32),
            jax.ShapeDtypeStruct((N, DIM), jnp.float32),
            jax.ShapeDtypeStruct((NC, N, HALF), jnp.float32),
        ],
    )(z.reshape(N, 1), node_table, W_r1T_0, W_n1T_0, b_n1_0)


def _lvl_body(n2_ref, hp_ref, rin_ref, wn2_ref, bn2_ref, wn3_ref,
              bn3_ref, wr1_ref, wn1_ref, bn1_ref,
              h_ref, racc_ref, hn_ref):
    node = jnp.concatenate([n2_ref[0], n2_ref[1]], axis=1)
    x1 = jnp.dot(node, wn2_ref[...], preferred_element_type=jnp.float32) + bn2_ref[0]
    a = _sp(x1, 0.5, 14.0)
    h = hp_ref[...] + jnp.dot(a, wn3_ref[...], preferred_element_type=jnp.float32) + bn3_ref[0]
    h_ref[...] = h
    racc_ref[...] = rin_ref[...] + jnp.dot(h, wr1_ref[...], preferred_element_type=jnp.float32)
    hn = jnp.dot(h, wn1_ref[...], preferred_element_type=jnp.float32) + bn1_ref[0]
    hn_ref[0] = hn[:, :HALF]
    hn_ref[1] = hn[:, HALF:]


def _node_level(node2, h_prev, racc, wn2T, bn2, wn3T, bn3,
                wr1T_next, wn1T_next, bn1_next, BN):
    N = h_prev.shape[0]
    nb = N // BN
    wspec = pl.BlockSpec((DIM, DIM), lambda i: (0, 0))
    bspec = pl.BlockSpec((1, DIM), lambda i: (0, 0))
    full = pl.BlockSpec((BN, DIM), lambda i: (i, 0))
    two = pl.BlockSpec((NC, BN, HALF), lambda i: (0, i, 0))
    return pl.pallas_call(
        _lvl_body,
        grid=(nb,),
        in_specs=[two, full, full, wspec, bspec, wspec, bspec,
                  wspec, wspec, bspec],
        out_specs=[full, full, two],
        out_shape=[
            jax.ShapeDtypeStruct((N, DIM), jnp.float32),
            jax.ShapeDtypeStruct((N, DIM), jnp.float32),
            jax.ShapeDtypeStruct((NC, N, HALF), jnp.float32),
        ],
    )(node2, h_prev, racc, wn2T, bn2, wn3T, bn3,
      wr1T_next, wn1T_next, bn1_next)


def _read_body(racc_ref, br1_ref, wr2_ref, o_ref):
    x = racc_ref[...] + br1_ref[0]
    h = _sp(x, 1.0, 20.0)
    p = jnp.sum(h * wr2_ref[...]).reshape(1, 1)

    @pl.when(pl.program_id(0) == 0)
    def _():
        o_ref[...] = jnp.zeros((1, 1), jnp.float32)

    o_ref[...] += p


def _readout(racc, b_r1, W_r2, BN):
    N = racc.shape[0]
    nb = N // BN
    return pl.pallas_call(
        _read_body,
        grid=(nb,),
        in_specs=[
            pl.BlockSpec((BN, DIM), lambda i: (i, 0)),
            pl.BlockSpec((1, DIM), lambda i: (0, 0)),
            pl.BlockSpec((1, DIM), lambda i: (0, 0)),
        ],
        out_specs=pl.BlockSpec((1, 1), lambda i: (0, 0)),
        out_shape=jax.ShapeDtypeStruct((1, 1), jnp.float32),
    )(racc, b_r1, W_r2)


def _make_sc_message(N, E, level):
    ept = E // NS
    nch = ept // CH
    stripe = ((N + NS - 1) // NS + 7) // 8 * 8
    mesh = plsc.VectorSubcoreMesh(core_axis_name="c", subcore_axis_name="s")

    @functools.partial(
        pl.kernel,
        mesh=mesh,
        compiler_params=pltpu.CompilerParams(use_tc_tiling_on_sc=False),
        out_type=jax.ShapeDtypeStruct((NC * N, HALF), jnp.float32),
        scratch_types=[
            pltpu.VMEM((CH,), jnp.int32),
            pltpu.VMEM((CH,), jnp.int32),
            pltpu.VMEM((CH, HALF), jnp.float32),
            pltpu.VMEM((CH, HALF), jnp.float32),
            pltpu.VMEM_SHARED((N, HALF), jnp.float32),
            pltpu.SemaphoreType.DMA,
        ],
    )
    def sck(src_hbm, dst_hbm, t_hbm, hn_hbm, zer_hbm, out_hbm,
            sidx, didx, gbuf, tbuf, acc, sem):
        c = lax.axis_index("c")
        s = lax.axis_index("s")
        sbase = pl.multiple_of(jnp.minimum(s * stripe, N - stripe), 8)
        pltpu.sync_copy(zer_hbm, acc.at[pl.ds(sbase, stripe)])
        plsc.subcore_barrier()

        ebase = s * ept
        toff = (2 * level + c) * E
        coff = c * N

        def chunk(k, _):
            base = ebase + k * CH
            pltpu.sync_copy(src_hbm.at[pl.ds(base, CH)], sidx)
            pltpu.sync_copy(dst_hbm.at[pl.ds(base, CH)], didx)
            def adj(j, _):
                sl = pl.ds(j * LANES, LANES)
                sidx[sl] = sidx[sl] + coff
                return 0
            lax.fori_loop(0, CH // LANES, adj, 0, unroll=True)
            pltpu.async_copy(hn_hbm.at[sidx], gbuf, sem).wait()
            pltpu.sync_copy(t_hbm.at[pl.ds(toff + base, CH)], tbuf)

            def rowfn(r, _):
                for hh in range(HALF // LANES):
                    sl = pl.ds(hh * LANES, LANES)
                    gbuf[r, sl] = (gbuf[r, sl] + 1.0) * tbuf[r, sl]
                return 0
            lax.fori_loop(0, CH, rowfn, 0)
            pltpu.sync_copy(gbuf, acc.at[didx], add=True)
            return 0

        lax.fori_loop(0, nch, chunk, 0)
        plsc.subcore_barrier()
        pltpu.sync_copy(acc.at[pl.ds(sbase, stripe)],
                        out_hbm.at[pl.ds(c * N + sbase, stripe)])

    return sck


def kernel(z, edge_index, dist, node_table, edge_table,
           W_n1, b_n1, W_e1, b_e1, W_l1, b_l1, W_l2, b_l2, W_l3, b_l3,
           W_n2, b_n2, W_n3, b_n3, W_r1, b_r1, W_r2, b_r2):
    N = z.shape[0]
    E = dist.shape[0]
    BE = 2000 if E % 2000 == 0 else E
    BN = 2000 if N % 2000 == 0 else N

    src = edge_index[0]
    dst = edge_index[1]

    W32 = jnp.einsum('lij,ljk->lik', W_l3, W_l2)
    b32 = jnp.einsum('lij,lj->li', W_l3, b_l2) + b_l3
    W_l1T = W_l1.transpose(0, 2, 1)
    W32T = W32.transpose(0, 2, 1)
    W_n1T = W_n1.transpose(0, 2, 1)
    W_n2T = W_n2.transpose(0, 2, 1)
    W_n3T = W_n3.transpose(0, 2, 1)
    W_r1T = W_r1.T

    t4 = _edge_t(dist, W_l1T, b_l1, W32T, b32, BE)
    t_flat = t4.reshape(N_CONV * NC * E, HALF)

    h, racc, hn2 = _node_init(
        z, node_table, W_r1T[:DIM], W_n1T[0], b_n1[0:1], BN)

    zer = jnp.zeros((((N + NS - 1) // NS + 7) // 8 * 8, HALF), jnp.float32)

    for l in range(N_CONV):
        sck = _make_sc_message(N, E, l)
        node_flat = sck(src, dst, t_flat, hn2.reshape(NC * N, HALF), zer)
        node2 = node_flat.reshape(NC, N, HALF)
        nxt = (l + 1) % N_CONV
        h, racc, hn2 = _node_level(
            node2, h, racc,
            W_n2T[l], b_n2[l:l + 1], W_n3T[l], b_n3[l:l + 1],
            W_r1T[DIM * (l + 1):DIM * (l + 2)], W_n1T[nxt], b_n1[nxt:nxt + 1],
            BN)

    out = _readout(racc, b_r1.reshape(1, DIM), W_r2, BN)
    return (out[0] + N * b_r2[0]).reshape(1)


# speedup vs baseline: 3.0672x; 1.8198x over previous
import functools

import jax
import jax.numpy as jnp
from jax import lax
from jax.experimental import pallas as pl
from jax.experimental.pallas import tpu as pltpu
from jax.experimental.pallas import tpu_sc as plsc

N_TYPES = 100
RBF_DIM = 30
DIM = 64
HALF = 32
N_CONV = 3
GAP = 10.0 / 29.0

NC = 2
NS = 16
LANES = 16
CH = 80


def _sp(x, beta, thr):
    xb = x * beta
    return jnp.where(xb > thr, x, (1.0 / beta) * jnp.log1p(jnp.exp(jnp.minimum(xb, thr))))


def _t_body(dist_ref, wl1_ref, bl1_ref, w32_ref, b32_ref, t_ref):
    d = dist_ref[...]
    j = lax.broadcasted_iota(jnp.int32, (1, RBF_DIM), 1).astype(jnp.float32)
    x = d - j * GAP
    rbf = jnp.exp(-(x * x) * (1.0 / GAP))
    for l in range(N_CONV):
        u = jnp.dot(rbf, wl1_ref[l], preferred_element_type=jnp.float32) + bl1_ref[l, 0]
        u = _sp(u, 0.5, 14.0)
        t = jnp.dot(u, w32_ref[l], preferred_element_type=jnp.float32) + b32_ref[l, 0]
        t_ref[l, 0] = t[:, :HALF]
        t_ref[l, 1] = t[:, HALF:]


def _edge_t(dist, W_l1T, b_l1, W32T, b32, BE):
    E = dist.shape[0]
    nb = E // BE
    return pl.pallas_call(
        _t_body,
        grid=(nb,),
        in_specs=[
            pl.BlockSpec((BE, 1), lambda i: (i, 0)),
            pl.BlockSpec((N_CONV, RBF_DIM, DIM), lambda i: (0, 0, 0)),
            pl.BlockSpec((N_CONV, 1, DIM), lambda i: (0, 0, 0)),
            pl.BlockSpec((N_CONV, DIM, DIM), lambda i: (0, 0, 0)),
            pl.BlockSpec((N_CONV, 1, DIM), lambda i: (0, 0, 0)),
        ],
        out_specs=pl.BlockSpec((N_CONV, NC, BE, HALF), lambda i: (0, 0, i, 0)),
        out_shape=jax.ShapeDtypeStruct((N_CONV, NC, E, HALF), jnp.float32),
    )(dist.reshape(E, 1), W_l1T, b_l1.reshape(N_CONV, 1, DIM),
      W32T, b32.reshape(N_CONV, 1, DIM))


def _init_body(z_ref, tbl_ref, wr1_ref, wn1_ref, bn1_ref,
               h0_ref, racc_ref, hn_ref):
    zb = z_ref[...]
    ids = lax.broadcasted_iota(jnp.int32, (zb.shape[0], N_TYPES), 1)
    oh = (ids == zb).astype(jnp.float32)
    h0 = jnp.dot(oh, tbl_ref[...], preferred_element_type=jnp.float32)
    h0_ref[...] = h0
    racc_ref[...] = jnp.dot(h0, wr1_ref[...], preferred_element_type=jnp.float32)
    hn = jnp.dot(h0, wn1_ref[...], preferred_element_type=jnp.float32) + bn1_ref[0]
    hn_ref[0] = hn[:, :HALF]
    hn_ref[1] = hn[:, HALF:]


def _node_init(z, node_table, W_r1T_0, W_n1T_0, b_n1_0, BN):
    N = z.shape[0]
    nb = N // BN
    return pl.pallas_call(
        _init_body,
        grid=(nb,),
        in_specs=[
            pl.BlockSpec((BN, 1), lambda i: (i, 0)),
            pl.BlockSpec((N_TYPES, DIM), lambda i: (0, 0)),
            pl.BlockSpec((DIM, DIM), lambda i: (0, 0)),
            pl.BlockSpec((DIM, DIM), lambda i: (0, 0)),
            pl.BlockSpec((1, DIM), lambda i: (0, 0)),
        ],
        out_specs=[
            pl.BlockSpec((BN, DIM), lambda i: (i, 0)),
            pl.BlockSpec((BN, DIM), lambda i: (i, 0)),
            pl.BlockSpec((NC, BN, HALF), lambda i: (0, i, 0)),
        ],
        out_shape=[
            jax.ShapeDtypeStruct((N, DIM), jnp.float32),
            jax.ShapeDtypeStruct((N, DIM), jnp.float32),
            jax.ShapeDtypeStruct((NC, N, HALF), jnp.float32),
        ],
    )(z.reshape(N, 1), node_table, W_r1T_0, W_n1T_0, b_n1_0)


def _lvl_body(n2_ref, hp_ref, rin_ref, wn2_ref, bn2_ref, wn3_ref,
              bn3_ref, wr1_ref, wn1_ref, bn1_ref,
              h_ref, racc_ref, hn_ref):
    node = jnp.concatenate([n2_ref[0], n2_ref[1]], axis=1)
    x1 = jnp.dot(node, wn2_ref[...], preferred_element_type=jnp.float32) + bn2_ref[0]
    a = _sp(x1, 0.5, 14.0)
    h = hp_ref[...] + jnp.dot(a, wn3_ref[...], preferred_element_type=jnp.float32) + bn3_ref[0]
    h_ref[...] = h
    racc_ref[...] = rin_ref[...] + jnp.dot(h, wr1_ref[...], preferred_element_type=jnp.float32)
    hn = jnp.dot(h, wn1_ref[...], preferred_element_type=jnp.float32) + bn1_ref[0]
    hn_ref[0] = hn[:, :HALF]
    hn_ref[1] = hn[:, HALF:]


def _node_level(node2, h_prev, racc, wn2T, bn2, wn3T, bn3,
                wr1T_next, wn1T_next, bn1_next, BN):
    N = h_prev.shape[0]
    nb = N // BN
    wspec = pl.BlockSpec((DIM, DIM), lambda i: (0, 0))
    bspec = pl.BlockSpec((1, DIM), lambda i: (0, 0))
    full = pl.BlockSpec((BN, DIM), lambda i: (i, 0))
    two = pl.BlockSpec((NC, BN, HALF), lambda i: (0, i, 0))
    return pl.pallas_call(
        _lvl_body,
        grid=(nb,),
        in_specs=[two, full, full, wspec, bspec, wspec, bspec,
                  wspec, wspec, bspec],
        out_specs=[full, full, two],
        out_shape=[
            jax.ShapeDtypeStruct((N, DIM), jnp.float32),
            jax.ShapeDtypeStruct((N, DIM), jnp.float32),
            jax.ShapeDtypeStruct((NC, N, HALF), jnp.float32),
        ],
    )(node2, h_prev, racc, wn2T, bn2, wn3T, bn3,
      wr1T_next, wn1T_next, bn1_next)


def _read_body(racc_ref, br1_ref, wr2_ref, o_ref):
    x = racc_ref[...] + br1_ref[0]
    h = _sp(x, 1.0, 20.0)
    p = jnp.sum(h * wr2_ref[...]).reshape(1, 1)

    @pl.when(pl.program_id(0) == 0)
    def _():
        o_ref[...] = jnp.zeros((1, 1), jnp.float32)

    o_ref[...] += p


def _readout(racc, b_r1, W_r2, BN):
    N = racc.shape[0]
    nb = N // BN
    return pl.pallas_call(
        _read_body,
        grid=(nb,),
        in_specs=[
            pl.BlockSpec((BN, DIM), lambda i: (i, 0)),
            pl.BlockSpec((1, DIM), lambda i: (0, 0)),
            pl.BlockSpec((1, DIM), lambda i: (0, 0)),
        ],
        out_specs=pl.BlockSpec((1, 1), lambda i: (0, 0)),
        out_shape=jax.ShapeDtypeStruct((1, 1), jnp.float32),
    )(racc, b_r1, W_r2)


def _make_sc_message(N, E, level):
    ept = E // NS
    nch = ept // CH
    stripe = ((N + NS - 1) // NS + 7) // 8 * 8
    mesh = plsc.VectorSubcoreMesh(core_axis_name="c", subcore_axis_name="s")

    NB = 3

    @functools.partial(
        pl.kernel,
        mesh=mesh,
        compiler_params=pltpu.CompilerParams(use_tc_tiling_on_sc=False),
        out_type=jax.ShapeDtypeStruct((NC * N, HALF), jnp.float32),
        scratch_types=(
            [pltpu.VMEM((CH,), jnp.int32) for _ in range(NB)] +
            [pltpu.VMEM((CH,), jnp.int32) for _ in range(NB)] +
            [pltpu.VMEM((CH, HALF), jnp.float32) for _ in range(NB)] +
            [pltpu.VMEM((CH, HALF), jnp.float32) for _ in range(NB)] +
            [pltpu.VMEM_SHARED((N, HALF), jnp.float32)] +
            [pltpu.SemaphoreType.DMA for _ in range(4 * NB)]
        ),
    )
    def sck(src_hbm, dst_hbm, t_hbm, hn_hbm, zer_hbm, out_hbm, *scr):
        sidx = scr[0:NB]
        didx = scr[NB:2 * NB]
        gbuf = scr[2 * NB:3 * NB]
        tbuf = scr[3 * NB:4 * NB]
        acc = scr[4 * NB]
        semi = scr[4 * NB + 1:4 * NB + 1 + NB]
        semt = scr[4 * NB + 1 + NB:4 * NB + 1 + 2 * NB]
        semg = scr[4 * NB + 1 + 2 * NB:4 * NB + 1 + 3 * NB]
        semsc = scr[4 * NB + 1 + 3 * NB:4 * NB + 1 + 4 * NB]

        c = lax.axis_index("c")
        s = lax.axis_index("s")
        sbase = pl.multiple_of(jnp.minimum(s * stripe, N - stripe), 8)
        pltpu.sync_copy(zer_hbm, acc.at[pl.ds(sbase, stripe)])
        plsc.subcore_barrier()

        ebase = s * ept
        toff = (2 * level + c) * E
        soff = c * E

        def issue_idx(k, b):
            base = ebase + k * CH
            pltpu.async_copy(src_hbm.at[pl.ds(soff + base, CH)], sidx[b], semi[b])
            pltpu.async_copy(dst_hbm.at[pl.ds(base, CH)], didx[b], semi[b])
            pltpu.async_copy(t_hbm.at[pl.ds(toff + base, CH)], tbuf[b], semt[b])

        def wait_idx(b):
            pltpu.make_async_copy(src_hbm.at[pl.ds(0, CH)], sidx[b], semi[b]).wait()
            pltpu.make_async_copy(dst_hbm.at[pl.ds(0, CH)], didx[b], semi[b]).wait()

        issue_idx(0, 0)
        issue_idx(1, 1)
        wait_idx(0)
        pltpu.async_copy(hn_hbm.at[sidx[0]], gbuf[0], semg[0])

        def body(k, b):
            b1 = (b + 1) % NB
            b2 = (b + 2) % NB
            @pl.when(k + 1 < nch)
            def _():
                wait_idx(b1)
                pltpu.async_copy(hn_hbm.at[sidx[b1]], gbuf[b1], semg[b1])

            pltpu.make_async_copy(hn_hbm.at[sidx[b]], gbuf[b], semg[b]).wait()
            pltpu.make_async_copy(t_hbm.at[pl.ds(0, CH)], tbuf[b], semt[b]).wait()
            g, t = gbuf[b], tbuf[b]
            for r in range(CH):
                for hh in range(HALF // LANES):
                    sl = pl.ds(hh * LANES, LANES)
                    g[r, sl] = (g[r, sl] + 1.0) * t[r, sl]
            pltpu.async_copy(gbuf[b], acc.at[didx[b]], semsc[b], add=True)

            @pl.when(k + 2 < nch)
            def _():
                @pl.when(k >= 1)
                def _():
                    pltpu.make_async_copy(gbuf[b2], acc.at[didx[b2]],
                                          semsc[b2]).wait()
                issue_idx(k + 2, b2)

        def outer(j, _):
            k3 = j * NB
            for b in range(NB):
                body(k3 + b, b)
            return 0

        nfull = nch // NB
        lax.fori_loop(0, nfull, outer, 0)
        for kk in range(nfull * NB, nch):
            body(kk, kk % NB)
        for kk in range(nch - NB, nch):
            b = kk % NB
            pltpu.make_async_copy(gbuf[b], acc.at[didx[b]], semsc[b]).wait()

        plsc.subcore_barrier()
        pltpu.sync_copy(acc.at[pl.ds(sbase, stripe)],
                        out_hbm.at[pl.ds(c * N + sbase, stripe)])

    return sck


def kernel(z, edge_index, dist, node_table, edge_table,
           W_n1, b_n1, W_e1, b_e1, W_l1, b_l1, W_l2, b_l2, W_l3, b_l3,
           W_n2, b_n2, W_n3, b_n3, W_r1, b_r1, W_r2, b_r2):
    N = z.shape[0]
    E = dist.shape[0]
    BE = 2000 if E % 2000 == 0 else E
    BN = 2000 if N % 2000 == 0 else N

    src = edge_index[0]
    dst = edge_index[1]
    src_pair = jnp.concatenate([src, src + N])

    W32 = jnp.einsum('lij,ljk->lik', W_l3, W_l2)
    b32 = jnp.einsum('lij,lj->li', W_l3, b_l2) + b_l3
    W_l1T = W_l1.transpose(0, 2, 1)
    W32T = W32.transpose(0, 2, 1)
    W_n1T = W_n1.transpose(0, 2, 1)
    W_n2T = W_n2.transpose(0, 2, 1)
    W_n3T = W_n3.transpose(0, 2, 1)
    W_r1T = W_r1.T

    t4 = _edge_t(dist, W_l1T, b_l1, W32T, b32, BE)
    t_flat = t4.reshape(N_CONV * NC * E, HALF)

    h, racc, hn2 = _node_init(
        z, node_table, W_r1T[:DIM], W_n1T[0], b_n1[0:1], BN)

    zer = jnp.zeros((((N + NS - 1) // NS + 7) // 8 * 8, HALF), jnp.float32)

    for l in range(N_CONV):
        sck = _make_sc_message(N, E, l)
        node_flat = sck(src_pair, dst, t_flat, hn2.reshape(NC * N, HALF), zer)
        node2 = node_flat.reshape(NC, N, HALF)
        nxt = (l + 1) % N_CONV
        h, racc, hn2 = _node_level(
            node2, h, racc,
            W_n2T[l], b_n2[l:l + 1], W_n3T[l], b_n3[l:l + 1],
            W_r1T[DIM * (l + 1):DIM * (l + 2)], W_n1T[nxt], b_n1[nxt:nxt + 1],
            BN)

    out = _readout(racc, b_r1.reshape(1, DIM), W_r2, BN)
    return (out[0] + N * b_r2[0]).reshape(1)


# speedup vs baseline: 4.1364x; 1.3486x over previous
import functools

import jax
import jax.numpy as jnp
from jax import lax
from jax.experimental import pallas as pl
from jax.experimental.pallas import tpu as pltpu
from jax.experimental.pallas import tpu_sc as plsc

N_TYPES = 100
RBF_DIM = 30
DIM = 64
HALF = 32
N_CONV = 3
GAP = 10.0 / 29.0

NC = 2
NS = 16
LANES = 16
CH = 128


def _sp(x, beta, thr):
    xb = x * beta
    return jnp.where(xb > thr, x, (1.0 / beta) * jnp.log1p(jnp.exp(jnp.minimum(xb, thr))))


def _t_body(d0_ref, d1_ref, d2_ref, d3_ref, wl1_ref, bl1_ref, w32_ref,
            b32_ref, t_ref):
    j_iota = lax.broadcasted_iota(jnp.int32, (1, RBF_DIM), 1).astype(jnp.float32)
    for j, dref in enumerate((d0_ref, d1_ref, d2_ref, d3_ref)):
        d = dref[...]
        x = d - j_iota * GAP
        rbf = jnp.exp(-(x * x) * (1.0 / GAP))
        for l in range(N_CONV):
            u = jnp.dot(rbf, wl1_ref[l], preferred_element_type=jnp.float32) + bl1_ref[l, 0]
            u = _sp(u, 0.5, 14.0)
            t = jnp.dot(u, w32_ref[l], preferred_element_type=jnp.float32) + b32_ref[l, 0]
            t_ref[l, 0, :, HALF * j:HALF * (j + 1)] = t[:, :HALF]
            t_ref[l, 1, :, HALF * j:HALF * (j + 1)] = t[:, HALF:]


def _edge_t(dist, W_l1T, b_l1, W32T, b32):
    E = dist.shape[0]
    EQ = E // 4
    BQ = 2000
    nb = EQ // BQ
    d2 = dist.reshape(E, 1)
    qspec = [pl.BlockSpec((BQ, 1), (lambda j: (lambda i: (i + j * nb, 0)))(j))
             for j in range(4)]
    return pl.pallas_call(
        _t_body,
        grid=(nb,),
        in_specs=qspec + [
            pl.BlockSpec((N_CONV, RBF_DIM, DIM), lambda i: (0, 0, 0)),
            pl.BlockSpec((N_CONV, 1, DIM), lambda i: (0, 0, 0)),
            pl.BlockSpec((N_CONV, DIM, DIM), lambda i: (0, 0, 0)),
            pl.BlockSpec((N_CONV, 1, DIM), lambda i: (0, 0, 0)),
        ],
        out_specs=pl.BlockSpec((N_CONV, NC, BQ, 4 * HALF), lambda i: (0, 0, i, 0)),
        out_shape=jax.ShapeDtypeStruct((N_CONV, NC, EQ, 4 * HALF), jnp.float32),
    )(d2, d2, d2, d2, W_l1T, b_l1.reshape(N_CONV, 1, DIM),
      W32T, b32.reshape(N_CONV, 1, DIM))


def _init_body(z_ref, tbl_ref, wr1_ref, wn1_ref, bn1_ref,
               h0_ref, racc_ref, hn_ref):
    zb = z_ref[...]
    ids = lax.broadcasted_iota(jnp.int32, (zb.shape[0], N_TYPES), 1)
    oh = (ids == zb).astype(jnp.float32)
    h0 = jnp.dot(oh, tbl_ref[...], preferred_element_type=jnp.float32)
    h0_ref[...] = h0
    racc_ref[...] = jnp.dot(h0, wr1_ref[...], preferred_element_type=jnp.float32)
    hn = jnp.dot(h0, wn1_ref[...], preferred_element_type=jnp.float32) + bn1_ref[0]
    hn_ref[0] = hn[:, :HALF]
    hn_ref[1] = hn[:, HALF:]


def _node_init(z, node_table, W_r1T_0, W_n1T_0, b_n1_0, BN):
    N = z.shape[0]
    nb = N // BN
    return pl.pallas_call(
        _init_body,
        grid=(nb,),
        in_specs=[
            pl.BlockSpec((BN, 1), lambda i: (i, 0)),
            pl.BlockSpec((N_TYPES, DIM), lambda i: (0, 0)),
            pl.BlockSpec((DIM, DIM), lambda i: (0, 0)),
            pl.BlockSpec((DIM, DIM), lambda i: (0, 0)),
            pl.BlockSpec((1, DIM), lambda i: (0, 0)),
        ],
        out_specs=[
            pl.BlockSpec((BN, DIM), lambda i: (i, 0)),
            pl.BlockSpec((BN, DIM), lambda i: (i, 0)),
            pl.BlockSpec((NC, BN, HALF), lambda i: (0, i, 0)),
        ],
        out_shape=[
            jax.ShapeDtypeStruct((N, DIM), jnp.float32),
            jax.ShapeDtypeStruct((N, DIM), jnp.float32),
            jax.ShapeDtypeStruct((NC, N, HALF), jnp.float32),
        ],
    )(z.reshape(N, 1), node_table, W_r1T_0, W_n1T_0, b_n1_0)


def _lvl_body(n2_ref, hp_ref, rin_ref, wn2_ref, bn2_ref, wn3_ref,
              bn3_ref, wr1_ref, wn1_ref, bn1_ref,
              h_ref, racc_ref, hn_ref):
    node = jnp.concatenate([n2_ref[0], n2_ref[1]], axis=1)
    x1 = jnp.dot(node, wn2_ref[...], preferred_element_type=jnp.float32) + bn2_ref[0]
    a = _sp(x1, 0.5, 14.0)
    h = hp_ref[...] + jnp.dot(a, wn3_ref[...], preferred_element_type=jnp.float32) + bn3_ref[0]
    h_ref[...] = h
    racc_ref[...] = rin_ref[...] + jnp.dot(h, wr1_ref[...], preferred_element_type=jnp.float32)
    hn = jnp.dot(h, wn1_ref[...], preferred_element_type=jnp.float32) + bn1_ref[0]
    hn_ref[0] = hn[:, :HALF]
    hn_ref[1] = hn[:, HALF:]


def _node_level(node2, h_prev, racc, wn2T, bn2, wn3T, bn3,
                wr1T_next, wn1T_next, bn1_next, BN):
    N = h_prev.shape[0]
    nb = N // BN
    wspec = pl.BlockSpec((DIM, DIM), lambda i: (0, 0))
    bspec = pl.BlockSpec((1, DIM), lambda i: (0, 0))
    full = pl.BlockSpec((BN, DIM), lambda i: (i, 0))
    two = pl.BlockSpec((NC, BN, HALF), lambda i: (0, i, 0))
    return pl.pallas_call(
        _lvl_body,
        grid=(nb,),
        in_specs=[two, full, full, wspec, bspec, wspec, bspec,
                  wspec, wspec, bspec],
        out_specs=[full, full, two],
        out_shape=[
            jax.ShapeDtypeStruct((N, DIM), jnp.float32),
            jax.ShapeDtypeStruct((N, DIM), jnp.float32),
            jax.ShapeDtypeStruct((NC, N, HALF), jnp.float32),
        ],
    )(node2, h_prev, racc, wn2T, bn2, wn3T, bn3,
      wr1T_next, wn1T_next, bn1_next)


def _read_body(racc_ref, br1_ref, wr2_ref, o_ref):
    x = racc_ref[...] + br1_ref[0]
    h = _sp(x, 1.0, 20.0)
    p = jnp.sum(h * wr2_ref[...]).reshape(1, 1)

    @pl.when(pl.program_id(0) == 0)
    def _():
        o_ref[...] = jnp.zeros((1, 1), jnp.float32)

    o_ref[...] += p


def _readout(racc, b_r1, W_r2, BN):
    N = racc.shape[0]
    nb = N // BN
    return pl.pallas_call(
        _read_body,
        grid=(nb,),
        in_specs=[
            pl.BlockSpec((BN, DIM), lambda i: (i, 0)),
            pl.BlockSpec((1, DIM), lambda i: (0, 0)),
            pl.BlockSpec((1, DIM), lambda i: (0, 0)),
        ],
        out_specs=pl.BlockSpec((1, 1), lambda i: (0, 0)),
        out_shape=jax.ShapeDtypeStruct((1, 1), jnp.float32),
    )(racc, b_r1, W_r2)


def _make_sc_message(N, E, level):
    EQ = E // 4
    nch = (E // CH + NS - 1) // NS
    ep = nch * NS * CH
    stripe = ((N + NS - 1) // NS + 7) // 8 * 8
    mesh = plsc.VectorSubcoreMesh(core_axis_name="c", subcore_axis_name="s")

    NB = 3

    TR = CH // 4

    @functools.partial(
        pl.kernel,
        mesh=mesh,
        compiler_params=pltpu.CompilerParams(use_tc_tiling_on_sc=False),
        out_type=jax.ShapeDtypeStruct((NC * N, HALF), jnp.float32),
        scratch_types=(
            [pltpu.VMEM((4, CH // 4), jnp.int32) for _ in range(NB)] +
            [pltpu.VMEM((4, CH // 4), jnp.int32) for _ in range(NB)] +
            [pltpu.VMEM((CH, HALF), jnp.float32) for _ in range(NB)] +
            [pltpu.VMEM((TR, 4 * HALF), jnp.float32) for _ in range(NB)] +
            [pltpu.VMEM_SHARED((N + 8, HALF), jnp.float32)] +
            [pltpu.SemaphoreType.DMA for _ in range(4 * NB)]
        ),
    )
    def sck(ei_hbm, t_hbm, hn_hbm, zer_hbm, out_hbm, *scr):
        sidx = scr[0:NB]
        didx = scr[NB:2 * NB]
        gbuf = scr[2 * NB:3 * NB]
        tbuf = scr[3 * NB:4 * NB]
        acc = scr[4 * NB]
        semi = scr[4 * NB + 1:4 * NB + 1 + NB]
        semt = scr[4 * NB + 1 + NB:4 * NB + 1 + 2 * NB]
        semg = scr[4 * NB + 1 + 2 * NB:4 * NB + 1 + 3 * NB]
        semsc = scr[4 * NB + 1 + 3 * NB:4 * NB + 1 + 4 * NB]

        c = lax.axis_index("c")
        s = lax.axis_index("s")
        sbase = pl.multiple_of(jnp.minimum(s * stripe, N - stripe), 8)
        pltpu.sync_copy(zer_hbm, acc.at[pl.ds(sbase, stripe)])
        plsc.subcore_barrier()

        rbase = s * nch * TR
        trbase = (2 * level + c) * EQ
        coff = c * N

        def _rclamp(k):
            r = jnp.minimum(rbase + k * TR, EQ - TR)
            return pl.multiple_of(r, 8)

        def _real(k):
            return rbase + k * TR < EQ

        def issue_idx(k, b):
            base = _rclamp(k)
            pltpu.async_copy(ei_hbm.at[0, :, pl.ds(base, TR)], sidx[b], semi[b])
            pltpu.async_copy(ei_hbm.at[1, :, pl.ds(base, TR)], didx[b], semi[b])
            pltpu.async_copy(t_hbm.at[pl.ds(trbase + base, TR)], tbuf[b], semt[b])

        def wait_idx(b):
            pltpu.make_async_copy(ei_hbm.at[0, :, pl.ds(0, TR)], sidx[b], semi[b]).wait()
            pltpu.make_async_copy(ei_hbm.at[1, :, pl.ds(0, TR)], didx[b], semi[b]).wait()
            for j in range(4):
                for q in range(TR // LANES):
                    sidx[b][j, pl.ds(q * LANES, LANES)] = (
                        sidx[b][j, pl.ds(q * LANES, LANES)] + coff)

        def issue_gather(b):
            for j in range(4):
                pltpu.async_copy(hn_hbm.at[sidx[b].at[j]],
                                 gbuf[b].at[pl.ds(TR * j, TR)], semg[b])

        def wait_gather(b):
            for j in range(4):
                pltpu.make_async_copy(hn_hbm.at[sidx[b].at[j]],
                                      gbuf[b].at[pl.ds(TR * j, TR)], semg[b]).wait()

        def issue_scatter(b):
            for j in range(4):
                pltpu.async_copy(gbuf[b].at[pl.ds(TR * j, TR)],
                                 acc.at[didx[b].at[j]], semsc[b], add=True)

        def wait_scatter(b):
            for j in range(4):
                pltpu.make_async_copy(gbuf[b].at[pl.ds(TR * j, TR)],
                                      acc.at[didx[b].at[j]], semsc[b]).wait()

        issue_idx(0, 0)
        issue_idx(1, 1)
        wait_idx(0)
        issue_gather(0)

        def body(k, b):
            b1 = (b + 1) % NB
            b2 = (b + 2) % NB
            @pl.when(k + 1 < nch)
            def _():
                wait_idx(b1)
                issue_gather(b1)

            wait_gather(b)
            pltpu.make_async_copy(t_hbm.at[pl.ds(0, TR)], tbuf[b], semt[b]).wait()
            g, t = gbuf[b], tbuf[b]

            def rowfn(r, _):
                for j in range(4):
                    for hh in range(HALF // LANES):
                        g[TR * j + r, pl.ds(hh * LANES, LANES)] = (
                            g[TR * j + r, pl.ds(hh * LANES, LANES)]
                            * t[r, pl.ds(HALF * j + hh * LANES, LANES)])
                return 0
            lax.fori_loop(0, TR, rowfn, 0, unroll=8)

            @pl.when(_real(k))
            def _():
                issue_scatter(b)

            @pl.when(k + 2 < nch)
            def _():
                @pl.when(jnp.logical_and(k >= 1, _real(k - 1)))
                def _():
                    wait_scatter(b2)
                issue_idx(k + 2, b2)

        def outer(j, _):
            k3 = j * NB
            for b in range(NB):
                body(k3 + b, b)
            return 0

        nfull = nch // NB
        lax.fori_loop(0, nfull, outer, 0)
        for kk in range(nfull * NB, nch):
            body(kk, kk % NB)
        for kk in range(nch - NB, nch):
            @pl.when(_real(kk))
            def _(kk=kk):
                wait_scatter(kk % NB)

        plsc.subcore_barrier()
        pltpu.sync_copy(acc.at[pl.ds(sbase, stripe)],
                        out_hbm.at[pl.ds(c * N + sbase, stripe)])

    return sck


def kernel(z, edge_index, dist, node_table, edge_table,
           W_n1, b_n1, W_e1, b_e1, W_l1, b_l1, W_l2, b_l2, W_l3, b_l3,
           W_n2, b_n2, W_n3, b_n3, W_r1, b_r1, W_r2, b_r2):
    N = z.shape[0]
    E = dist.shape[0]
    BE = 2000 if E % 2000 == 0 else E
    BN = 2000 if N % 2000 == 0 else N

    src = edge_index[0]
    dst = edge_index[1]

    ei4 = edge_index.reshape(2, 4, E // 4)

    W32 = jnp.einsum('lij,ljk->lik', W_l3, W_l2)
    b32 = jnp.einsum('lij,lj->li', W_l3, b_l2) + b_l3
    W_l1T = W_l1.transpose(0, 2, 1)
    W32T = W32.transpose(0, 2, 1)
    W_n1T = W_n1.transpose(0, 2, 1)
    W_n2T = W_n2.transpose(0, 2, 1)
    W_n3T = W_n3.transpose(0, 2, 1)
    W_r1T = W_r1.T

    t4 = _edge_t(dist, W_l1T, b_l1, W32T, b32)
    t_flat = t4.reshape(N_CONV * NC * (E // 4), 4 * HALF)

    b_n1p = b_n1 + 1.0
    h, racc, hn2 = _node_init(
        z, node_table, W_r1T[:DIM], W_n1T[0], b_n1p[0:1], BN)

    zer = jnp.zeros((((N + NS - 1) // NS + 7) // 8 * 8, HALF), jnp.float32)

    for l in range(N_CONV):
        sck = _make_sc_message(N, E, l)
        node_flat = sck(ei4, t_flat, hn2.reshape(NC * N, HALF), zer)
        node2 = node_flat.reshape(NC, N, HALF)
        nxt = (l + 1) % N_CONV
        h, racc, hn2 = _node_level(
            node2, h, racc,
            W_n2T[l], b_n2[l:l + 1], W_n3T[l], b_n3[l:l + 1],
            W_r1T[DIM * (l + 1):DIM * (l + 2)], W_n1T[nxt], b_n1p[nxt:nxt + 1],
            BN)

    out = _readout(racc, b_r1.reshape(1, DIM), W_r2, BN)
    return (out[0] + N * b_r2[0]).reshape(1)
